# baseline (speedup 1.0000x reference)
import jax
import jax.numpy as jnp
from jax.experimental import pallas as pl
from jax.experimental.pallas import tpu as pltpu

_BS = 2
_S = 4
_N = 256
_DV = 512
_DM = 256
_INNER = 512
_H = 8
_DH = _INNER // _H
_G = _BS * _S
_R = _G * _N
_L = _S * _N
_NEG = -1e30
_SCALE = _DH ** -0.5


def _fused_kernel(va_ref, vb_ref, vc_ref, vd_ref,
                  ma_ref, mb_ref, mc_ref, md_ref,
                  wqkv_ref, winner_ref, wouter_ref, wout_ref, bout_ref,
                  out_ref,
                  qb_s, kb_s, vaug_s, bias_s, means_s,
                  wqkv_bf, winner_bf, wout_bf):
    b = pl.program_id(0)
    j = pl.program_id(1)

    @pl.when(jnp.logical_and(b == 0, j == 0))
    def _():
        wqkv_bf[...] = wqkv_ref[...].astype(jnp.bfloat16)
        winner_bf[...] = winner_ref[...].astype(jnp.bfloat16)
        wout_bf[...] = wout_ref[...].astype(jnp.bfloat16)

    @pl.when(j == 0)
    def _():
        vals_refs = (va_ref, vb_ref, vc_ref, vd_ref)
        meta_refs = (ma_ref, mb_ref, mc_ref, md_ref)
        ones = jnp.ones((_N, 1), jnp.float32)
        zeros = jnp.zeros((_N, _DH - 1), jnp.float32)
        for src in range(_S):
            x = vals_refs[src][0].astype(jnp.bfloat16)
            qkv = jnp.dot(x, wqkv_bf[...],
                          preferred_element_type=jnp.float32)
            m = meta_refs[src][0]
            mp = jnp.dot(m.astype(jnp.bfloat16), winner_bf[...],
                         preferred_element_type=jnp.float32)
            qp = jnp.clip(qkv[:, :_INNER], -5.0, 5.0) * _SCALE
            kp = jnp.clip(qkv[:, _INNER:2 * _INNER], -5.0, 5.0)
            v = qkv[:, 2 * _INNER:]
            qm2 = jnp.clip(mp[:, :_INNER], -5.0, 5.0) * _SCALE
            km2 = jnp.clip(mp[:, _INNER:], -5.0, 5.0)
            qbs, kbs, vas = [], [], []
            for h in range(_H):
                sl = slice(h * _DH, (h + 1) * _DH)
                qbs += [qp[:, sl], qm2[:, sl]]
                kbs += [kp[:, sl], km2[:, sl]]
                vas += [v[:, sl], ones, zeros]
            row = pl.ds(src * _N, _N)
            qb_s[row, :] = jnp.concatenate(qbs, axis=1).astype(jnp.bfloat16)
            kb_s[row, :] = jnp.concatenate(kbs, axis=1).astype(jnp.bfloat16)
            vaug_s[row, :] = jnp.concatenate(vas, axis=1).astype(jnp.bfloat16)
            means_s[pl.ds(src, 1), :] = jnp.mean(m, axis=0, keepdims=True)

        mm = means_s[0:_S, :]
        qk = jnp.dot(mm, wouter_ref[...],
                     preferred_element_type=jnp.float32)
        qm = jnp.clip(qk[:, :_INNER], -5.0, 5.0)
        km = jnp.clip(qk[:, _INNER:], -5.0, 5.0)
        dots = jax.lax.dot_general(
            qm, km, (((1,), (1,)), ((), ())),
            preferred_element_type=jnp.float32) * (_INNER ** -0.5)
        mx = jnp.max(dots, axis=1, keepdims=True)
        p = jnp.exp(dots - mx)
        sm = p / jnp.sum(p, axis=1, keepdims=True)
        rows = jax.lax.broadcasted_iota(jnp.int32, (_S, _S), 0)
        cols = jax.lax.broadcasted_iota(jnp.int32, (_S, _S), 1)
        attn_v = sm + 2.0 * (cols == rows).astype(jnp.float32)
        colv = [attn_v[:, c:c + 1] for c in range(_S)]
        bias_cols = []
        for c in range(_S):
            rank = jnp.zeros((_S, 1), jnp.int32)
            for k in range(_S):
                if k == c:
                    continue
                beats = colv[k] > colv[c]
                if k < c:
                    beats = beats | (colv[k] == colv[c])
                rank = rank + beats.astype(jnp.int32)
            bias_cols.append(jnp.where(rank < _S - 1, 0.0, _NEG))
        key_src = jax.lax.broadcasted_iota(jnp.int32, (_S, _L), 1) // _N
        bias_s[...] = jnp.where(
            key_src == 0, bias_cols[0],
            jnp.where(key_src == 1, bias_cols[1],
                      jnp.where(key_src == 2, bias_cols[2], bias_cols[3])))

    @pl.when(j > 0)
    def _():
        s = j - 1
        qrow = pl.ds(s * _N, _N)
        bias_row = bias_s[pl.ds(s, 1), :]
        ohs = []
        for h in range(_H):
            sl = slice(h * 2 * _DH, (h + 1) * 2 * _DH)
            scoresf = jax.lax.dot_general(
                qb_s[qrow, sl], kb_s[:, sl], (((1,), (1,)), ((), ())),
                preferred_element_type=jnp.float32)
            gm = jnp.max(jnp.max(scoresf, axis=0, keepdims=True))
            adj = (bias_row - gm).astype(jnp.bfloat16)
            p = jnp.exp(scoresf.astype(jnp.bfloat16) + adj)
            res = jnp.dot(p, vaug_s[:, sl],
                          preferred_element_type=jnp.float32)
            oh = res[:, :_DH] * (1.0 / res[:, _DH:_DH + 1])
            ohs.append(oh.astype(jnp.bfloat16))
        omerged = jnp.concatenate(ohs, axis=1)
        out_ref[...] = jnp.dot(
            omerged, wout_bf[...],
            preferred_element_type=jnp.float32) + bout_ref[...]


def kernel(values_a, values_b, values_c, values_d,
           metadata_a, metadata_b, metadata_c, metadata_d,
           W_meta_qk_outer, W_values_qkv, W_meta_qk_inner, W_out, b_out):
    b_out2 = b_out.reshape(1, _DV)

    def _vspec():
        return pl.BlockSpec((1, _N, _DV), lambda b, j: (b, 0, 0))

    def _mspec():
        return pl.BlockSpec((1, _N, _DM), lambda b, j: (b, 0, 0))

    out = pl.pallas_call(
        _fused_kernel,
        grid=(_BS, _S + 1),
        in_specs=[
            _vspec(), _vspec(), _vspec(), _vspec(),
            _mspec(), _mspec(), _mspec(), _mspec(),
            pl.BlockSpec((_DV, 3 * _INNER), lambda b, j: (0, 0)),
            pl.BlockSpec((_DM, 2 * _INNER), lambda b, j: (0, 0)),
            pl.BlockSpec((_DM, 2 * _INNER), lambda b, j: (0, 0)),
            pl.BlockSpec((_INNER, _DV), lambda b, j: (0, 0)),
            pl.BlockSpec((1, _DV), lambda b, j: (0, 0)),
        ],
        out_specs=pl.BlockSpec(
            (_N, _DV), lambda b, j: (b * _S + jnp.maximum(j, 1) - 1, 0)),
        out_shape=jax.ShapeDtypeStruct((_R, _DV), jnp.float32),
        scratch_shapes=[
            pltpu.VMEM((_L, 2 * _INNER), jnp.bfloat16),
            pltpu.VMEM((_L, 2 * _INNER), jnp.bfloat16),
            pltpu.VMEM((_L, 2 * _INNER), jnp.bfloat16),
            pltpu.VMEM((_S, _L), jnp.float32),
            pltpu.VMEM((_G, _DM), jnp.float32),
            pltpu.VMEM((_DV, 3 * _INNER), jnp.bfloat16),
            pltpu.VMEM((_DM, 2 * _INNER), jnp.bfloat16),
            pltpu.VMEM((_INNER, _DV), jnp.bfloat16),
        ],
    )(values_a, values_b, values_c, values_d,
      metadata_a, metadata_b, metadata_c, metadata_d,
      W_values_qkv, W_meta_qk_inner, W_meta_qk_outer, W_out, b_out2)

    return out.reshape(_BS, _S, _N, _DV)


# speedup vs baseline: 1.2569x; 1.2569x over previous
import jax
import jax.numpy as jnp
from jax.experimental import pallas as pl
from jax.experimental.pallas import tpu as pltpu

_BS = 2
_S = 4
_N = 256
_DV = 512
_DM = 256
_INNER = 512
_H = 8
_DH = _INNER // _H
_G = _BS * _S
_R = _G * _N
_L = _S * _N
_NEG = -1e30
_SCALE = _DH ** -0.5


def _fused_kernel(va_ref, vb_ref, vc_ref, vd_ref,
                  ma_ref, mb_ref, mc_ref, md_ref,
                  wqkv_ref, winner_ref, wouter_ref, wout_ref, bout_ref,
                  out_ref,
                  qb_s, kb_s, vaug_s, excl_s, means_s,
                  wqkv_bf, winner_bf, wout_bf):
    b = pl.program_id(0)
    j = pl.program_id(1)

    @pl.when(jnp.logical_and(b == 0, j == 0))
    def _():
        wqkv_bf[...] = wqkv_ref[...].astype(jnp.bfloat16)
        winner_bf[...] = winner_ref[...].astype(jnp.bfloat16)
        wout_bf[...] = wout_ref[...].astype(jnp.bfloat16)

    @pl.when(j == 0)
    def _():
        vals_refs = (va_ref, vb_ref, vc_ref, vd_ref)
        meta_refs = (ma_ref, mb_ref, mc_ref, md_ref)
        ones = jnp.ones((_N, 1), jnp.float32)
        zeros = jnp.zeros((_N, _DH - 1), jnp.float32)
        for src in range(_S):
            x = vals_refs[src][0].astype(jnp.bfloat16)
            qkv = jnp.dot(x, wqkv_bf[...],
                          preferred_element_type=jnp.float32)
            m = meta_refs[src][0]
            mp = jnp.dot(m.astype(jnp.bfloat16), winner_bf[...],
                         preferred_element_type=jnp.float32)
            qp = jnp.clip(qkv[:, :_INNER], -5.0, 5.0) * _SCALE
            kp = jnp.clip(qkv[:, _INNER:2 * _INNER], -5.0, 5.0)
            v = qkv[:, 2 * _INNER:]
            qm2 = jnp.clip(mp[:, :_INNER], -5.0, 5.0) * _SCALE
            km2 = jnp.clip(mp[:, _INNER:], -5.0, 5.0)
            qbs, kbs, vas = [], [], []
            for h in range(_H):
                sl = slice(h * _DH, (h + 1) * _DH)
                qbs += [qp[:, sl], qm2[:, sl]]
                kbs += [kp[:, sl], km2[:, sl]]
                vas += [v[:, sl], ones, zeros]
            row = pl.ds(src * _N, _N)
            qb_s[row, :] = jnp.concatenate(qbs, axis=1).astype(jnp.bfloat16)
            kb_s[row, :] = jnp.concatenate(kbs, axis=1).astype(jnp.bfloat16)
            vaug_s[row, :] = jnp.concatenate(vas, axis=1).astype(jnp.bfloat16)
            means_s[pl.ds(src, 1), :] = jnp.mean(m, axis=0, keepdims=True)

        mm = means_s[0:_S, :]
        qk = jnp.dot(mm, wouter_ref[...],
                     preferred_element_type=jnp.float32)
        qm = jnp.clip(qk[:, :_INNER], -5.0, 5.0)
        km = jnp.clip(qk[:, _INNER:], -5.0, 5.0)
        dots = jax.lax.dot_general(
            qm, km, (((1,), (1,)), ((), ())),
            preferred_element_type=jnp.float32) * (_INNER ** -0.5)
        mx = jnp.max(dots, axis=1, keepdims=True)
        p = jnp.exp(dots - mx)
        sm = p / jnp.sum(p, axis=1, keepdims=True)
        rows = jax.lax.broadcasted_iota(jnp.int32, (_S, _S), 0)
        cols = jax.lax.broadcasted_iota(jnp.int32, (_S, _S), 1)
        attn_v = sm + 2.0 * (cols == rows).astype(jnp.float32)
        colv = [attn_v[:, c:c + 1] for c in range(_S)]
        excl = jnp.zeros((_S, 1), jnp.int32)
        for c in range(_S):
            rank = jnp.zeros((_S, 1), jnp.int32)
            for k in range(_S):
                if k == c:
                    continue
                beats = colv[k] > colv[c]
                if k < c:
                    beats = beats | (colv[k] == colv[c])
                rank = rank + beats.astype(jnp.int32)
            excl = excl + c * (rank == _S - 1).astype(jnp.int32)
        excl_s[0:_S, :] = jnp.broadcast_to(excl, (_S, 128))

    @pl.when(j > 0)
    def _():
        s = j - 1
        qrow = pl.ds(s * _N, _N)
        e = excl_s[s, 0]
        offs = [(w + (w >= e).astype(jnp.int32)) * _N for w in range(_S - 1)]
        ohs = []
        for h in range(_H):
            sl = slice(h * 2 * _DH, (h + 1) * 2 * _DH)
            q = qb_s[qrow, sl]
            scb = []
            for w in range(_S - 1):
                krow = pl.ds(offs[w], _N)
                scw = jax.lax.dot_general(
                    q, kb_s[krow, sl], (((1,), (1,)), ((), ())),
                    preferred_element_type=jnp.float32)
                scb.append(scw.astype(jnp.bfloat16))
            mx = jnp.maximum(
                jnp.maximum(jnp.max(scb[0], axis=1, keepdims=True),
                            jnp.max(scb[1], axis=1, keepdims=True)),
                jnp.max(scb[2], axis=1, keepdims=True))
            res = jnp.zeros((_N, 2 * _DH), jnp.float32)
            for w in range(_S - 1):
                p = jnp.exp(scb[w] - mx)
                res = res + jnp.dot(p, vaug_s[pl.ds(offs[w], _N), sl],
                                    preferred_element_type=jnp.float32)
            oh = res[:, :_DH] * (1.0 / res[:, _DH:_DH + 1])
            ohs.append(oh.astype(jnp.bfloat16))
        omerged = jnp.concatenate(ohs, axis=1)
        out_ref[...] = jnp.dot(
            omerged, wout_bf[...],
            preferred_element_type=jnp.float32) + bout_ref[...]


def kernel(values_a, values_b, values_c, values_d,
           metadata_a, metadata_b, metadata_c, metadata_d,
           W_meta_qk_outer, W_values_qkv, W_meta_qk_inner, W_out, b_out):
    b_out2 = b_out.reshape(1, _DV)

    def _vspec():
        return pl.BlockSpec((1, _N, _DV), lambda b, j: (b, 0, 0))

    def _mspec():
        return pl.BlockSpec((1, _N, _DM), lambda b, j: (b, 0, 0))

    out = pl.pallas_call(
        _fused_kernel,
        grid=(_BS, _S + 1),
        in_specs=[
            _vspec(), _vspec(), _vspec(), _vspec(),
            _mspec(), _mspec(), _mspec(), _mspec(),
            pl.BlockSpec((_DV, 3 * _INNER), lambda b, j: (0, 0)),
            pl.BlockSpec((_DM, 2 * _INNER), lambda b, j: (0, 0)),
            pl.BlockSpec((_DM, 2 * _INNER), lambda b, j: (0, 0)),
            pl.BlockSpec((_INNER, _DV), lambda b, j: (0, 0)),
            pl.BlockSpec((1, _DV), lambda b, j: (0, 0)),
        ],
        out_specs=pl.BlockSpec(
            (_N, _DV), lambda b, j: (b * _S + jnp.maximum(j, 1) - 1, 0)),
        out_shape=jax.ShapeDtypeStruct((_R, _DV), jnp.float32),
        scratch_shapes=[
            pltpu.VMEM((_L, 2 * _INNER), jnp.bfloat16),
            pltpu.VMEM((_L, 2 * _INNER), jnp.bfloat16),
            pltpu.VMEM((_L, 2 * _INNER), jnp.bfloat16),
            pltpu.VMEM((_G, 128), jnp.int32),
            pltpu.VMEM((_G, _DM), jnp.float32),
            pltpu.VMEM((_DV, 3 * _INNER), jnp.bfloat16),
            pltpu.VMEM((_DM, 2 * _INNER), jnp.bfloat16),
            pltpu.VMEM((_INNER, _DV), jnp.bfloat16),
        ],
    )(values_a, values_b, values_c, values_d,
      metadata_a, metadata_b, metadata_c, metadata_d,
      W_values_qkv, W_meta_qk_inner, W_meta_qk_outer, W_out, b_out2)

    return out.reshape(_BS, _S, _N, _DV)


# speedup vs baseline: 1.2907x; 1.0269x over previous
import jax
import jax.numpy as jnp
from jax.experimental import pallas as pl
from jax.experimental.pallas import tpu as pltpu

_BS = 2
_S = 4
_N = 256
_DV = 512
_DM = 256
_INNER = 512
_H = 8
_DH = _INNER // _H
_G = _BS * _S
_R = _G * _N
_L = _S * _N
_NEG = -1e30
_SCALE = _DH ** -0.5


def _fused_kernel(va_ref, vb_ref, vc_ref, vd_ref,
                  ma_ref, mb_ref, mc_ref, md_ref,
                  wqkv_ref, winner_ref, wouter_ref, wout_ref, bout_ref,
                  out_ref,
                  qb_s, kb_s, vaug_s, excl_s, means_s,
                  wqkv_bf, winner_bf, wout_bf):
    b = pl.program_id(0)
    j = pl.program_id(1)

    @pl.when(jnp.logical_and(b == 0, j == 0))
    def _():
        wqkv_bf[...] = wqkv_ref[...].astype(jnp.bfloat16)
        winner_bf[...] = winner_ref[...].astype(jnp.bfloat16)
        wout_bf[...] = wout_ref[...].astype(jnp.bfloat16)

    @pl.when(j == 0)
    def _():
        vals_refs = (va_ref, vb_ref, vc_ref, vd_ref)
        meta_refs = (ma_ref, mb_ref, mc_ref, md_ref)
        ones = jnp.ones((_N, 1), jnp.bfloat16)
        zeros = jnp.zeros((_N, _DH - 1), jnp.bfloat16)
        for src in range(_S):
            x = vals_refs[src][0].astype(jnp.bfloat16)
            qkv = jnp.dot(x, wqkv_bf[...],
                          preferred_element_type=jnp.float32
                          ).astype(jnp.bfloat16)
            m = meta_refs[src][0]
            mp = jnp.dot(m.astype(jnp.bfloat16), winner_bf[...],
                         preferred_element_type=jnp.float32
                         ).astype(jnp.bfloat16)
            qp = jnp.clip(qkv[:, :_INNER], -5.0, 5.0) * _SCALE
            kp = jnp.clip(qkv[:, _INNER:2 * _INNER], -5.0, 5.0)
            v = qkv[:, 2 * _INNER:]
            qm2 = jnp.clip(mp[:, :_INNER], -5.0, 5.0) * _SCALE
            km2 = jnp.clip(mp[:, _INNER:], -5.0, 5.0)
            qbs, kbs, vas = [], [], []
            for h in range(_H):
                sl = slice(h * _DH, (h + 1) * _DH)
                qbs += [qp[:, sl], qm2[:, sl]]
                kbs += [kp[:, sl], km2[:, sl]]
                vas += [v[:, sl], ones, zeros]
            row = pl.ds(src * _N, _N)
            qb_s[row, :] = jnp.concatenate(qbs, axis=1)
            kb_s[row, :] = jnp.concatenate(kbs, axis=1)
            vaug_s[row, :] = jnp.concatenate(vas, axis=1)
            means_s[pl.ds(src, 1), :] = jnp.mean(m, axis=0, keepdims=True)

        mm = means_s[0:_S, :]
        qk = jnp.dot(mm, wouter_ref[...],
                     preferred_element_type=jnp.float32)
        qm = jnp.clip(qk[:, :_INNER], -5.0, 5.0)
        km = jnp.clip(qk[:, _INNER:], -5.0, 5.0)
        dots = jax.lax.dot_general(
            qm, km, (((1,), (1,)), ((), ())),
            preferred_element_type=jnp.float32) * (_INNER ** -0.5)
        mx = jnp.max(dots, axis=1, keepdims=True)
        p = jnp.exp(dots - mx)
        sm = p / jnp.sum(p, axis=1, keepdims=True)
        rows = jax.lax.broadcasted_iota(jnp.int32, (_S, _S), 0)
        cols = jax.lax.broadcasted_iota(jnp.int32, (_S, _S), 1)
        attn_v = sm + 2.0 * (cols == rows).astype(jnp.float32)
        colv = [attn_v[:, c:c + 1] for c in range(_S)]
        excl = jnp.zeros((_S, 1), jnp.int32)
        for c in range(_S):
            rank = jnp.zeros((_S, 1), jnp.int32)
            for k in range(_S):
                if k == c:
                    continue
                beats = colv[k] > colv[c]
                if k < c:
                    beats = beats | (colv[k] == colv[c])
                rank = rank + beats.astype(jnp.int32)
            excl = excl + c * (rank == _S - 1).astype(jnp.int32)
        excl_s[0:_S, :] = jnp.broadcast_to(excl, (_S, 128))

    @pl.when(j > 0)
    def _():
        s = j - 1
        qrow = pl.ds(s * _N, _N)
        e = excl_s[s, 0]
        offs = [(w + (w >= e).astype(jnp.int32)) * _N for w in range(_S - 1)]
        ohs = []
        for h in range(_H):
            sl = slice(h * 2 * _DH, (h + 1) * 2 * _DH)
            q = qb_s[qrow, sl]
            scb = []
            for w in range(_S - 1):
                krow = pl.ds(offs[w], _N)
                scw = jax.lax.dot_general(
                    q, kb_s[krow, sl], (((1,), (1,)), ((), ())),
                    preferred_element_type=jnp.float32)
                scb.append(scw.astype(jnp.bfloat16))
            mx = jnp.maximum(
                jnp.maximum(jnp.max(scb[0], axis=1, keepdims=True),
                            jnp.max(scb[1], axis=1, keepdims=True)),
                jnp.max(scb[2], axis=1, keepdims=True))
            res = jnp.zeros((_N, 2 * _DH), jnp.float32)
            for w in range(_S - 1):
                p = jnp.exp(scb[w] - mx)
                res = res + jnp.dot(p, vaug_s[pl.ds(offs[w], _N), sl],
                                    preferred_element_type=jnp.float32)
            oh = res[:, :_DH] * (1.0 / res[:, _DH:_DH + 1])
            ohs.append(oh.astype(jnp.bfloat16))
        omerged = jnp.concatenate(ohs, axis=1)
        out_ref[...] = jnp.dot(
            omerged, wout_bf[...],
            preferred_element_type=jnp.float32) + bout_ref[...]


def kernel(values_a, values_b, values_c, values_d,
           metadata_a, metadata_b, metadata_c, metadata_d,
           W_meta_qk_outer, W_values_qkv, W_meta_qk_inner, W_out, b_out):
    b_out2 = b_out.reshape(1, _DV)

    def _vspec():
        return pl.BlockSpec((1, _N, _DV), lambda b, j: (b, 0, 0))

    def _mspec():
        return pl.BlockSpec((1, _N, _DM), lambda b, j: (b, 0, 0))

    out = pl.pallas_call(
        _fused_kernel,
        grid=(_BS, _S + 1),
        in_specs=[
            _vspec(), _vspec(), _vspec(), _vspec(),
            _mspec(), _mspec(), _mspec(), _mspec(),
            pl.BlockSpec((_DV, 3 * _INNER), lambda b, j: (0, 0)),
            pl.BlockSpec((_DM, 2 * _INNER), lambda b, j: (0, 0)),
            pl.BlockSpec((_DM, 2 * _INNER), lambda b, j: (0, 0)),
            pl.BlockSpec((_INNER, _DV), lambda b, j: (0, 0)),
            pl.BlockSpec((1, _DV), lambda b, j: (0, 0)),
        ],
        out_specs=pl.BlockSpec(
            (_N, _DV), lambda b, j: (b * _S + jnp.maximum(j, 1) - 1, 0)),
        out_shape=jax.ShapeDtypeStruct((_R, _DV), jnp.float32),
        scratch_shapes=[
            pltpu.VMEM((_L, 2 * _INNER), jnp.bfloat16),
            pltpu.VMEM((_L, 2 * _INNER), jnp.bfloat16),
            pltpu.VMEM((_L, 2 * _INNER), jnp.bfloat16),
            pltpu.VMEM((_G, 128), jnp.int32),
            pltpu.VMEM((_G, _DM), jnp.float32),
            pltpu.VMEM((_DV, 3 * _INNER), jnp.bfloat16),
            pltpu.VMEM((_DM, 2 * _INNER), jnp.bfloat16),
            pltpu.VMEM((_INNER, _DV), jnp.bfloat16),
        ],
    )(values_a, values_b, values_c, values_d,
      metadata_a, metadata_b, metadata_c, metadata_d,
      W_values_qkv, W_meta_qk_inner, W_meta_qk_outer, W_out, b_out2)

    return out.reshape(_BS, _S, _N, _DV)
